# Initial kernel scaffold; baseline (speedup 1.0000x reference)
import jax
import jax.numpy as jnp
from jax.experimental import pallas as pl


def kernel(node_states, edge_lists, W_msg, b_msg, w_ih, w_hh, b_ih, b_hh):
    raise NotImplementedError("write your pallas kernel here")


# speedup vs baseline: 3.9163x; 3.9163x over previous
import functools

import jax
import jax.numpy as jnp
from jax import lax
from jax.experimental import pallas as pl
from jax.experimental.pallas import tpu as pltpu
from jax.experimental.pallas import tpu_sc as plsc

N = 10000
H = 128
E = 320000
TIMESTEPS = 4

NC = 2
NS = 16
NW = NC * NS
CHUNK = 128
CH_PER_TILE = -(-E // (NW * CHUNK))
E_PAD = NW * CH_PER_TILE * CHUNK
N_PAD = 10240
ROWS_PER_TILE = N_PAD // NS

RB = 256
GRID = N_PAD // RB

def _edge_body(prop_hbm, src_hbm, tgt_hbm, zr_hbm, out_hbm, srcv, tgtv, rows,
               acc, sem):
    c = lax.axis_index("c")
    s = lax.axis_index("s")
    wid = s * NC + c
    r0 = s * ROWS_PER_TILE
    pltpu.sync_copy(zr_hbm, acc.at[pl.ds(r0, ROWS_PER_TILE)])
    plsc.subcore_barrier()
    base = wid * (CH_PER_TILE * CHUNK)

    def body(i, _):
        off = base + i * CHUNK
        pltpu.sync_copy(src_hbm.at[pl.ds(off, CHUNK)], srcv)
        pltpu.sync_copy(tgt_hbm.at[pl.ds(off, CHUNK)], tgtv)
        pltpu.async_copy(prop_hbm.at[srcv], rows, sem).wait()
        pltpu.sync_copy(rows, acc.at[tgtv], add=True)
        return 0

    lax.fori_loop(0, CH_PER_TILE, body, 0)
    plsc.subcore_barrier()
    pltpu.sync_copy(acc.at[pl.ds(r0, ROWS_PER_TILE)],
                    out_hbm.at[c, pl.ds(r0, ROWS_PER_TILE)])


@functools.lru_cache(maxsize=None)
def _edge_kernel_fn():
    mesh = plsc.VectorSubcoreMesh(core_axis_name="c", subcore_axis_name="s",
                                  num_cores=NC, num_subcores=NS)
    return pl.kernel(
        _edge_body,
        out_type=jax.ShapeDtypeStruct((NC, N_PAD, H), jnp.float32),
        mesh=mesh,
        scratch_types=[
            pltpu.VMEM((CHUNK,), jnp.int32),
            pltpu.VMEM((CHUNK,), jnp.int32),
            pltpu.VMEM((CHUNK, H), jnp.float32),
            pltpu.VMEM_SHARED((N_PAD, H), jnp.float32),
            pltpu.SemaphoreType.DMA,
        ],
    )


def _edge_kernel(*args):
    return _edge_kernel_fn()(*args)


def _mm_body(h_ref, wT_ref, bc_ref, prop_ref, gh_ref):
    y = jnp.dot(h_ref[...], wT_ref[...], preferred_element_type=jnp.float32)
    y = y + bc_ref[0:1, :]
    prop_ref[...] = y[:, :H]
    gh_ref[...] = y[:, H:]


_mm = pl.pallas_call(
    _mm_body,
    grid=(GRID,),
    in_specs=[
        pl.BlockSpec((RB, H), lambda i: (i, 0)),
        pl.BlockSpec((H, 4 * H), lambda i: (0, 0)),
        pl.BlockSpec((8, 4 * H), lambda i: (0, 0)),
    ],
    out_specs=[
        pl.BlockSpec((RB, H), lambda i: (i, 0)),
        pl.BlockSpec((RB, 3 * H), lambda i: (i, 0)),
    ],
    out_shape=[
        jax.ShapeDtypeStruct((N_PAD, H), jnp.float32),
        jax.ShapeDtypeStruct((N_PAD, 3 * H), jnp.float32),
    ],
)


def _gru_body(p_ref, cnt_ref, gh_ref, h_ref, wT_ref, bih_ref, out_ref):
    p = p_ref[0] + p_ref[1]
    cnt = cnt_ref[0] + cnt_ref[1]
    x = p / jnp.maximum(cnt, 1.0)
    gi = jnp.dot(x, wT_ref[...], preferred_element_type=jnp.float32)
    gi = gi + bih_ref[0:1, :]
    gh = gh_ref[...]
    r = jax.nn.sigmoid(gi[:, :H] + gh[:, :H])
    z = jax.nn.sigmoid(gi[:, H:2 * H] + gh[:, H:2 * H])
    n = jnp.tanh(gi[:, 2 * H:] + r * gh[:, 2 * H:])
    out_ref[...] = (1.0 - z) * n + z * h_ref[...]


_gru = pl.pallas_call(
    _gru_body,
    grid=(GRID,),
    in_specs=[
        pl.BlockSpec((NC, RB, H), lambda i: (0, i, 0)),
        pl.BlockSpec((NC, RB, H), lambda i: (0, i, 0)),
        pl.BlockSpec((RB, 3 * H), lambda i: (i, 0)),
        pl.BlockSpec((RB, H), lambda i: (i, 0)),
        pl.BlockSpec((H, 3 * H), lambda i: (0, 0)),
        pl.BlockSpec((8, 3 * H), lambda i: (0, 0)),
    ],
    out_specs=pl.BlockSpec((RB, H), lambda i: (i, 0)),
    out_shape=jax.ShapeDtypeStruct((N_PAD, H), jnp.float32),
)


def kernel(node_states, edge_lists, W_msg, b_msg, w_ih, w_hh, b_ih, b_hh):
    f32 = jnp.float32
    el = edge_lists[0]
    src = el[:, 0].astype(jnp.int32)
    tgt = el[:, 1].astype(jnp.int32)
    pad = E_PAD - E
    pad_idx = N + (jnp.arange(pad, dtype=jnp.int32) % (N_PAD - N))
    srcp = jnp.concatenate([src, pad_idx])
    tgtp = jnp.concatenate([tgt, pad_idx])

    h = jnp.zeros((N_PAD, H), f32).at[:N].set(node_states)
    zrows = jnp.zeros((ROWS_PER_TILE, H), f32)
    onesmat = jnp.ones((N_PAD, H), f32)

    cnt_part = _edge_kernel(onesmat, srcp, tgtp, zrows)

    wT_cat = jnp.concatenate([W_msg, w_hh], axis=0).T
    bc = jnp.broadcast_to(
        jnp.concatenate([b_msg, b_hh])[None, :], (8, 4 * H))
    wT_ih = w_ih.T
    bih = jnp.broadcast_to(b_ih[None, :], (8, 3 * H))

    for _ in range(TIMESTEPS):
        prop, gh = _mm(h, wT_cat, bc)
        part = _edge_kernel(prop, srcp, tgtp, zrows)
        h = _gru(part, cnt_part, gh, h, wT_ih, bih)
    return h[:N], node_states


# speedup vs baseline: 6.1665x; 1.5746x over previous
import functools

import jax
import jax.numpy as jnp
from jax import lax
from jax.experimental import pallas as pl
from jax.experimental.pallas import tpu as pltpu
from jax.experimental.pallas import tpu_sc as plsc

N = 10000
H = 128
E = 320000
TIMESTEPS = 4

NC = 2
NS = 16
NW = NC * NS
CHUNK = 128
CH_PER_TILE = 80
BLK = 40
NBLK = CH_PER_TILE // BLK
E_PAD = NW * CH_PER_TILE * CHUNK
N_PAD = 10240
ROWS_PER_TILE = N_PAD // NS
CW = 16

RB = 256
GRID = N_PAD // RB


def _edge_body(prop_hbm, src_hbm, tgt_hbm, zr_hbm, out_hbm, srca, tgta,
               rows0, rows1, acc, g0, g1, s0, s1):
    c = lax.axis_index("c")
    s = lax.axis_index("s")
    wid = s * NC + c
    row0 = s * ROWS_PER_TILE
    pltpu.sync_copy(zr_hbm, acc.at[pl.ds(row0, ROWS_PER_TILE)])
    plsc.subcore_barrier()

    bufs = [(rows0, g0, s0), (rows1, g1, s1)]

    @pl.loop(0, NBLK)
    def _blk(blk):
        pltpu.sync_copy(src_hbm.at[wid, pl.ds(blk * BLK, BLK)], srca)
        pltpu.sync_copy(tgt_hbm.at[wid, pl.ds(blk * BLK, BLK)], tgta)
        pltpu.async_copy(prop_hbm.at[srca.at[0]], rows0, g0)
        for k in range(BLK):
            rb, gs, ss = bufs[k % 2]
            pltpu.make_async_copy(prop_hbm.at[srca.at[k]], rb, gs).wait()
            if k + 1 < BLK:
                ob, og, osem = bufs[(k + 1) % 2]
                if k >= 1:
                    pltpu.make_async_copy(ob, acc.at[tgta.at[k - 1]],
                                          osem).wait()
                pltpu.async_copy(prop_hbm.at[srca.at[k + 1]], ob, og)
            pltpu.async_copy(rb, acc.at[tgta.at[k]], ss, add=True)
        rb, _, ss = bufs[(BLK - 2) % 2]
        pltpu.make_async_copy(rb, acc.at[tgta.at[BLK - 2]], ss).wait()
        rb, _, ss = bufs[(BLK - 1) % 2]
        pltpu.make_async_copy(rb, acc.at[tgta.at[BLK - 1]], ss).wait()

    plsc.subcore_barrier()
    pltpu.sync_copy(acc.at[pl.ds(row0, ROWS_PER_TILE)],
                    out_hbm.at[c, pl.ds(row0, ROWS_PER_TILE)])


def _cnt_body(tgt_hbm, ones_hbm, z16_hbm, out_hbm, tgta, ones, acc16, csem):
    c = lax.axis_index("c")
    s = lax.axis_index("s")
    wid = s * NC + c
    row0 = s * ROWS_PER_TILE
    pltpu.sync_copy(z16_hbm, acc16.at[pl.ds(row0, ROWS_PER_TILE)])
    pltpu.sync_copy(tgt_hbm.at[wid], tgta)
    pltpu.sync_copy(ones_hbm, ones)
    plsc.subcore_barrier()

    def fire(i, _):
        pltpu.async_copy(ones, acc16.at[tgta.at[i]], csem, add=True)
        return 0

    lax.fori_loop(0, CH_PER_TILE, fire, 0)

    def drain(i, _):
        pltpu.make_async_copy(ones, acc16.at[tgta.at[i]], csem).wait()
        return 0

    lax.fori_loop(0, CH_PER_TILE, drain, 0)
    plsc.subcore_barrier()
    pltpu.sync_copy(acc16.at[pl.ds(row0, ROWS_PER_TILE)],
                    out_hbm.at[c, pl.ds(row0, ROWS_PER_TILE)])


@functools.lru_cache(maxsize=None)
def _sc_kernels():
    mesh = plsc.VectorSubcoreMesh(core_axis_name="c", subcore_axis_name="s",
                                  num_cores=NC, num_subcores=NS)
    edge = pl.kernel(
        _edge_body,
        out_type=jax.ShapeDtypeStruct((NC, N_PAD, H), jnp.float32),
        mesh=mesh,
        scratch_types=[
            pltpu.VMEM((BLK, CHUNK), jnp.int32),
            pltpu.VMEM((BLK, CHUNK), jnp.int32),
            pltpu.VMEM((CHUNK, H), jnp.float32),
            pltpu.VMEM((CHUNK, H), jnp.float32),
            pltpu.VMEM_SHARED((N_PAD, H), jnp.float32),
            pltpu.SemaphoreType.DMA,
            pltpu.SemaphoreType.DMA,
            pltpu.SemaphoreType.DMA,
            pltpu.SemaphoreType.DMA,
        ],
    )
    cnt = pl.kernel(
        _cnt_body,
        out_type=jax.ShapeDtypeStruct((NC, N_PAD, CW), jnp.float32),
        mesh=mesh,
        scratch_types=[
            pltpu.VMEM((CH_PER_TILE, CHUNK), jnp.int32),
            pltpu.VMEM((CHUNK, CW), jnp.float32),
            pltpu.VMEM_SHARED((N_PAD, CW), jnp.float32),
            pltpu.SemaphoreType.DMA,
        ],
    )
    return edge, cnt


def _mm_body(h_ref, wT_ref, bc_ref, prop_ref, gh_ref):
    y = jnp.dot(h_ref[...], wT_ref[...], preferred_element_type=jnp.float32)
    y = y + bc_ref[0:1, :]
    prop_ref[...] = y[:, :H]
    gh_ref[...] = y[:, H:]


_mm = pl.pallas_call(
    _mm_body,
    grid=(GRID,),
    in_specs=[
        pl.BlockSpec((RB, H), lambda i: (i, 0)),
        pl.BlockSpec((H, 4 * H), lambda i: (0, 0)),
        pl.BlockSpec((8, 4 * H), lambda i: (0, 0)),
    ],
    out_specs=[
        pl.BlockSpec((RB, H), lambda i: (i, 0)),
        pl.BlockSpec((RB, 3 * H), lambda i: (i, 0)),
    ],
    out_shape=[
        jax.ShapeDtypeStruct((N_PAD, H), jnp.float32),
        jax.ShapeDtypeStruct((N_PAD, 3 * H), jnp.float32),
    ],
)


def _gru_body(p_ref, cnt_ref, gh_ref, h_ref, wT_ref, bih_ref, out_ref):
    p = p_ref[0] + p_ref[1]
    cnt = cnt_ref[0, :, 0:1] + cnt_ref[1, :, 0:1]
    x = p / jnp.maximum(cnt, 1.0)
    gi = jnp.dot(x, wT_ref[...], preferred_element_type=jnp.float32)
    gi = gi + bih_ref[0:1, :]
    gh = gh_ref[...]
    r = jax.nn.sigmoid(gi[:, :H] + gh[:, :H])
    z = jax.nn.sigmoid(gi[:, H:2 * H] + gh[:, H:2 * H])
    n = jnp.tanh(gi[:, 2 * H:] + r * gh[:, 2 * H:])
    out_ref[...] = (1.0 - z) * n + z * h_ref[...]


_gru = pl.pallas_call(
    _gru_body,
    grid=(GRID,),
    in_specs=[
        pl.BlockSpec((NC, RB, H), lambda i: (0, i, 0)),
        pl.BlockSpec((NC, RB, CW), lambda i: (0, i, 0)),
        pl.BlockSpec((RB, 3 * H), lambda i: (i, 0)),
        pl.BlockSpec((RB, H), lambda i: (i, 0)),
        pl.BlockSpec((H, 3 * H), lambda i: (0, 0)),
        pl.BlockSpec((8, 3 * H), lambda i: (0, 0)),
    ],
    out_specs=pl.BlockSpec((RB, H), lambda i: (i, 0)),
    out_shape=jax.ShapeDtypeStruct((N_PAD, H), jnp.float32),
)


def kernel(node_states, edge_lists, W_msg, b_msg, w_ih, w_hh, b_ih, b_hh):
    f32 = jnp.float32
    el = edge_lists[0]
    src = el[:, 0].astype(jnp.int32)
    tgt = el[:, 1].astype(jnp.int32)
    pad = E_PAD - E
    pad_idx = N + (jnp.arange(pad, dtype=jnp.int32) % (N_PAD - N))
    srcp = jnp.concatenate([src, pad_idx]).reshape(NW, CH_PER_TILE, CHUNK)
    tgtp = jnp.concatenate([tgt, pad_idx]).reshape(NW, CH_PER_TILE, CHUNK)

    h = jnp.zeros((N_PAD, H), f32).at[:N].set(node_states)
    zrows = jnp.zeros((ROWS_PER_TILE, H), f32)
    z16 = jnp.zeros((ROWS_PER_TILE, CW), f32)
    ones16 = jnp.ones((CHUNK, CW), f32)

    edge_k, cnt_k = _sc_kernels()
    onesmat = jnp.ones((N_PAD, H), f32)
    cnt_part_full = edge_k(onesmat, srcp, tgtp, zrows)
    cnt_part = cnt_part_full[:, :, :CW]

    wT_cat = jnp.concatenate([W_msg, w_hh], axis=0).T
    bc = jnp.broadcast_to(
        jnp.concatenate([b_msg, b_hh])[None, :], (8, 4 * H))
    wT_ih = w_ih.T
    bih = jnp.broadcast_to(b_ih[None, :], (8, 3 * H))

    for _ in range(TIMESTEPS):
        prop, gh = _mm(h, wT_cat, bc)
        part = edge_k(prop, srcp, tgtp, zrows)
        h = _gru(part, cnt_part, gh, h, wT_ih, bih)
    return h[:N], node_states


# speedup vs baseline: 7.0965x; 1.1508x over previous
import functools

import jax
import jax.numpy as jnp
from jax import lax
from jax.experimental import pallas as pl
from jax.experimental.pallas import tpu as pltpu
from jax.experimental.pallas import tpu_sc as plsc

N = 10000
H = 128
E = 320000
TIMESTEPS = 4

NC = 2
NS = 16
NW = NC * NS
CHUNK = 128
CH_PER_TILE = 80
BLK = 40
NBLK = CH_PER_TILE // BLK
E_PAD = NW * CH_PER_TILE * CHUNK
N_PAD = 10240
ROWS_PER_TILE = N_PAD // NS
CW = 16

RB = 256
GRID = N_PAD // RB


def _edge_body(prop_hbm, src_hbm, tgt_hbm, zr_hbm, out_hbm, srca, tgta,
               rows0, rows1, acc, g0, g1, s0, s1):
    c = lax.axis_index("c")
    s = lax.axis_index("s")
    wid = s * NC + c
    row0 = s * ROWS_PER_TILE
    pltpu.sync_copy(zr_hbm, acc.at[pl.ds(row0, ROWS_PER_TILE)])
    plsc.subcore_barrier()

    bufs = [(rows0, g0, s0), (rows1, g1, s1)]

    @pl.loop(0, NBLK)
    def _blk(blk):
        pltpu.sync_copy(src_hbm.at[wid, pl.ds(blk * BLK, BLK)], srca)
        pltpu.sync_copy(tgt_hbm.at[wid, pl.ds(blk * BLK, BLK)], tgta)
        pltpu.async_copy(prop_hbm.at[srca.at[0]], rows0, g0)
        for k in range(BLK):
            rb, gs, ss = bufs[k % 2]
            pltpu.make_async_copy(prop_hbm.at[srca.at[k]], rb, gs).wait()
            if k + 1 < BLK:
                ob, og, osem = bufs[(k + 1) % 2]
                if k >= 1:
                    pltpu.make_async_copy(ob, acc.at[tgta.at[k - 1]],
                                          osem).wait()
                pltpu.async_copy(prop_hbm.at[srca.at[k + 1]], ob, og)
            pltpu.async_copy(rb, acc.at[tgta.at[k]], ss, add=True)
        rb, _, ss = bufs[(BLK - 2) % 2]
        pltpu.make_async_copy(rb, acc.at[tgta.at[BLK - 2]], ss).wait()
        rb, _, ss = bufs[(BLK - 1) % 2]
        pltpu.make_async_copy(rb, acc.at[tgta.at[BLK - 1]], ss).wait()

    plsc.subcore_barrier()
    pltpu.sync_copy(acc.at[pl.ds(row0, ROWS_PER_TILE)],
                    out_hbm.at[c, pl.ds(row0, ROWS_PER_TILE)])


def _cnt_body(tgt_hbm, ones_hbm, z1_hbm, out_hbm, tgta, ones, acc1, csem):
    c = lax.axis_index("c")
    s = lax.axis_index("s")
    wid = s * NC + c
    row0 = s * ROWS_PER_TILE
    pltpu.sync_copy(z1_hbm, acc1.at[pl.ds(row0, ROWS_PER_TILE)])
    pltpu.sync_copy(tgt_hbm.at[wid], tgta)
    pltpu.sync_copy(ones_hbm, ones)
    plsc.subcore_barrier()

    for i in range(CH_PER_TILE):
        pltpu.async_copy(ones, acc1.at[tgta.at[i]], csem, add=True)
    for i in range(CH_PER_TILE):
        pltpu.make_async_copy(ones, acc1.at[tgta.at[i]], csem).wait()
    plsc.subcore_barrier()
    pltpu.sync_copy(acc1.at[pl.ds(row0, ROWS_PER_TILE)],
                    out_hbm.at[c, pl.ds(row0, ROWS_PER_TILE)])


@functools.lru_cache(maxsize=None)
def _sc_kernels():
    mesh = plsc.VectorSubcoreMesh(core_axis_name="c", subcore_axis_name="s",
                                  num_cores=NC, num_subcores=NS)
    edge = pl.kernel(
        _edge_body,
        out_type=jax.ShapeDtypeStruct((NC, N_PAD, H), jnp.float32),
        mesh=mesh,
        scratch_types=[
            pltpu.VMEM((BLK, CHUNK), jnp.int32),
            pltpu.VMEM((BLK, CHUNK), jnp.int32),
            pltpu.VMEM((CHUNK, H), jnp.float32),
            pltpu.VMEM((CHUNK, H), jnp.float32),
            pltpu.VMEM_SHARED((N_PAD, H), jnp.float32),
            pltpu.SemaphoreType.DMA,
            pltpu.SemaphoreType.DMA,
            pltpu.SemaphoreType.DMA,
            pltpu.SemaphoreType.DMA,
        ],
    )
    cnt = pl.kernel(
        _cnt_body,
        out_type=jax.ShapeDtypeStruct((NC, N_PAD), jnp.float32),
        mesh=mesh,
        scratch_types=[
            pltpu.VMEM((CH_PER_TILE, CHUNK), jnp.int32),
            pltpu.VMEM((CHUNK,), jnp.float32),
            pltpu.VMEM_SHARED((N_PAD,), jnp.float32),
            pltpu.SemaphoreType.DMA,
        ],
    )
    return edge, cnt


def _mm_body(h_ref, wT_ref, bc_ref, prop_ref, gh_ref):
    y = jnp.dot(h_ref[...], wT_ref[...], preferred_element_type=jnp.float32)
    y = y + bc_ref[0:1, :]
    prop_ref[...] = y[:, :H]
    gh_ref[...] = y[:, H:]


_mm = pl.pallas_call(
    _mm_body,
    grid=(GRID,),
    in_specs=[
        pl.BlockSpec((RB, H), lambda i: (i, 0)),
        pl.BlockSpec((H, 4 * H), lambda i: (0, 0)),
        pl.BlockSpec((8, 4 * H), lambda i: (0, 0)),
    ],
    out_specs=[
        pl.BlockSpec((RB, H), lambda i: (i, 0)),
        pl.BlockSpec((RB, 3 * H), lambda i: (i, 0)),
    ],
    out_shape=[
        jax.ShapeDtypeStruct((N_PAD, H), jnp.float32),
        jax.ShapeDtypeStruct((N_PAD, 3 * H), jnp.float32),
    ],
)


def _gru_body(p_ref, cnt_ref, gh_ref, h_ref, wT_ref, bih_ref, out_ref):
    p = p_ref[0] + p_ref[1]
    cnt = (cnt_ref[0] + cnt_ref[1]).reshape(RB, 1)
    x = p / jnp.maximum(cnt, 1.0)
    gi = jnp.dot(x, wT_ref[...], preferred_element_type=jnp.float32)
    gi = gi + bih_ref[0:1, :]
    gh = gh_ref[...]
    r = jax.nn.sigmoid(gi[:, :H] + gh[:, :H])
    z = jax.nn.sigmoid(gi[:, H:2 * H] + gh[:, H:2 * H])
    n = jnp.tanh(gi[:, 2 * H:] + r * gh[:, 2 * H:])
    out_ref[...] = (1.0 - z) * n + z * h_ref[...]


_gru = pl.pallas_call(
    _gru_body,
    grid=(GRID,),
    in_specs=[
        pl.BlockSpec((NC, RB, H), lambda i: (0, i, 0)),
        pl.BlockSpec((NC, RB), lambda i: (0, i)),
        pl.BlockSpec((RB, 3 * H), lambda i: (i, 0)),
        pl.BlockSpec((RB, H), lambda i: (i, 0)),
        pl.BlockSpec((H, 3 * H), lambda i: (0, 0)),
        pl.BlockSpec((8, 3 * H), lambda i: (0, 0)),
    ],
    out_specs=pl.BlockSpec((RB, H), lambda i: (i, 0)),
    out_shape=jax.ShapeDtypeStruct((N_PAD, H), jnp.float32),
)


def kernel(node_states, edge_lists, W_msg, b_msg, w_ih, w_hh, b_ih, b_hh):
    f32 = jnp.float32
    el = edge_lists[0]
    src = el[:, 0].astype(jnp.int32)
    tgt = el[:, 1].astype(jnp.int32)
    pad = E_PAD - E
    pad_idx = N + (jnp.arange(pad, dtype=jnp.int32) % (N_PAD - N))
    srcp = jnp.concatenate([src, pad_idx]).reshape(NW, CH_PER_TILE, CHUNK)
    tgtp = jnp.concatenate([tgt, pad_idx]).reshape(NW, CH_PER_TILE, CHUNK)

    h = jnp.zeros((N_PAD, H), f32).at[:N].set(node_states)
    zrows = jnp.zeros((ROWS_PER_TILE, H), f32)
    z1 = jnp.zeros((ROWS_PER_TILE,), f32)
    ones1 = jnp.ones((CHUNK,), f32)

    edge_k, cnt_k = _sc_kernels()
    cnt_part = cnt_k(tgtp, ones1, z1)

    wT_cat = jnp.concatenate([W_msg, w_hh], axis=0).T
    bc = jnp.broadcast_to(
        jnp.concatenate([b_msg, b_hh])[None, :], (8, 4 * H))
    wT_ih = w_ih.T
    bih = jnp.broadcast_to(b_ih[None, :], (8, 3 * H))

    for _ in range(TIMESTEPS):
        prop, gh = _mm(h, wT_cat, bc)
        part = edge_k(prop, srcp, tgtp, zrows)
        h = _gru(part, cnt_part, gh, h, wT_ih, bih)
    return h[:N], node_states


# speedup vs baseline: 7.5150x; 1.0590x over previous
import functools

import jax
import jax.numpy as jnp
from jax import lax
from jax.experimental import pallas as pl
from jax.experimental.pallas import tpu as pltpu
from jax.experimental.pallas import tpu_sc as plsc

N = 10000
H = 128
E = 320000
TIMESTEPS = 4

NC = 2
NS = 16
NW = NC * NS
CHUNK = 128
CH_PER_TILE = 80
BLK = 40
NBLK = CH_PER_TILE // BLK
E_PAD = NW * CH_PER_TILE * CHUNK
N_PAD = 10240
ROWS_PER_TILE = N_PAD // NS

RB = 200
GRID = N // RB


def _edge_body(prop_hbm, src_hbm, tgt_hbm, zr_hbm, out_hbm, srca, tgta,
               rows0, rows1, acc, g0, g1, s0, s1):
    c = lax.axis_index("c")
    s = lax.axis_index("s")
    wid = s * NC + c
    row0 = s * ROWS_PER_TILE
    pltpu.sync_copy(zr_hbm, acc.at[pl.ds(row0, ROWS_PER_TILE)])
    plsc.subcore_barrier()

    bufs = [(rows0, g0, s0), (rows1, g1, s1)]

    @pl.loop(0, NBLK)
    def _blk(blk):
        pltpu.sync_copy(src_hbm.at[wid, pl.ds(blk * BLK, BLK)], srca)
        pltpu.sync_copy(tgt_hbm.at[wid, pl.ds(blk * BLK, BLK)], tgta)
        pltpu.async_copy(prop_hbm.at[srca.at[0]], rows0, g0)
        for k in range(BLK):
            rb, gs, ss = bufs[k % 2]
            pltpu.make_async_copy(prop_hbm.at[srca.at[k]], rb, gs).wait()
            if k + 1 < BLK:
                ob, og, osem = bufs[(k + 1) % 2]
                if k >= 1:
                    pltpu.make_async_copy(ob, acc.at[tgta.at[k - 1]],
                                          osem).wait()
                pltpu.async_copy(prop_hbm.at[srca.at[k + 1]], ob, og)
            pltpu.async_copy(rb, acc.at[tgta.at[k]], ss, add=True)
        rb, _, ss = bufs[(BLK - 2) % 2]
        pltpu.make_async_copy(rb, acc.at[tgta.at[BLK - 2]], ss).wait()
        rb, _, ss = bufs[(BLK - 1) % 2]
        pltpu.make_async_copy(rb, acc.at[tgta.at[BLK - 1]], ss).wait()

    plsc.subcore_barrier()
    pltpu.sync_copy(acc.at[pl.ds(row0, ROWS_PER_TILE)],
                    out_hbm.at[c, pl.ds(row0, ROWS_PER_TILE)])


def _cnt_body(tgt_hbm, ones_hbm, z1_hbm, out_hbm, tgta, ones, acc1, csem):
    c = lax.axis_index("c")
    s = lax.axis_index("s")
    wid = s * NC + c
    row0 = s * ROWS_PER_TILE
    pltpu.sync_copy(z1_hbm, acc1.at[pl.ds(row0, ROWS_PER_TILE)])
    pltpu.sync_copy(tgt_hbm.at[wid], tgta)
    pltpu.sync_copy(ones_hbm, ones)
    plsc.subcore_barrier()

    for i in range(CH_PER_TILE):
        pltpu.async_copy(ones, acc1.at[tgta.at[i]], csem, add=True)
    for i in range(CH_PER_TILE):
        pltpu.make_async_copy(ones, acc1.at[tgta.at[i]], csem).wait()
    plsc.subcore_barrier()
    pltpu.sync_copy(acc1.at[pl.ds(row0, ROWS_PER_TILE)],
                    out_hbm.at[c, pl.ds(row0, ROWS_PER_TILE)])


@functools.lru_cache(maxsize=None)
def _sc_kernels():
    mesh = plsc.VectorSubcoreMesh(core_axis_name="c", subcore_axis_name="s",
                                  num_cores=NC, num_subcores=NS)
    edge = pl.kernel(
        _edge_body,
        out_type=jax.ShapeDtypeStruct((NC, N_PAD, H), jnp.float32),
        mesh=mesh,
        scratch_types=[
            pltpu.VMEM((BLK, CHUNK), jnp.int32),
            pltpu.VMEM((BLK, CHUNK), jnp.int32),
            pltpu.VMEM((CHUNK, H), jnp.float32),
            pltpu.VMEM((CHUNK, H), jnp.float32),
            pltpu.VMEM_SHARED((N_PAD, H), jnp.float32),
            pltpu.SemaphoreType.DMA,
            pltpu.SemaphoreType.DMA,
            pltpu.SemaphoreType.DMA,
            pltpu.SemaphoreType.DMA,
        ],
    )
    cnt = pl.kernel(
        _cnt_body,
        out_type=jax.ShapeDtypeStruct((NC, N_PAD), jnp.float32),
        mesh=mesh,
        scratch_types=[
            pltpu.VMEM((CH_PER_TILE, CHUNK), jnp.int32),
            pltpu.VMEM((CHUNK,), jnp.float32),
            pltpu.VMEM_SHARED((N_PAD,), jnp.float32),
            pltpu.SemaphoreType.DMA,
        ],
    )
    return edge, cnt


def _gru_math(p_ref, cnt_ref, gh_ref, h_ref, wT_ih_ref, bih_ref):
    p = p_ref[0] + p_ref[1]
    cnt = cnt_ref[0] + cnt_ref[1]
    x = p / jnp.maximum(cnt, 1.0)
    gi = jnp.dot(x, wT_ih_ref[...], preferred_element_type=jnp.float32)
    gi = gi + bih_ref[0:1, :]
    gh = gh_ref[...]
    r = jax.nn.sigmoid(gi[:, :H] + gh[:, :H])
    z = jax.nn.sigmoid(gi[:, H:2 * H] + gh[:, H:2 * H])
    n = jnp.tanh(gi[:, 2 * H:] + r * gh[:, 2 * H:])
    return (1.0 - z) * n + z * h_ref[...]


def _mm_math(h, wT_cat_ref, bc_ref, prop_ref, gh_ref):
    y = jnp.dot(h, wT_cat_ref[...], preferred_element_type=jnp.float32)
    y = y + bc_ref[0:1, :]
    prop_ref[...] = y[:, :H]
    gh_ref[...] = y[:, H:]


def _mm_body(h_ref, wT_cat_ref, bc_ref, prop_ref, gh_ref):
    _mm_math(h_ref[...], wT_cat_ref, bc_ref, prop_ref, gh_ref)


def _gru_body(p_ref, cnt_ref, gh_ref, h_ref, wT_ih_ref, bih_ref, out_ref):
    out_ref[...] = _gru_math(p_ref, cnt_ref, gh_ref, h_ref, wT_ih_ref,
                             bih_ref)


def _gmm_body(p_ref, cnt_ref, gh_ref, h_ref, wT_ih_ref, bih_ref,
              wT_cat_ref, bc_ref, hn_ref, prop_ref, ghn_ref):
    hn = _gru_math(p_ref, cnt_ref, gh_ref, h_ref, wT_ih_ref, bih_ref)
    hn_ref[...] = hn
    _mm_math(hn, wT_cat_ref, bc_ref, prop_ref, ghn_ref)


_spec_part = pl.BlockSpec((NC, RB, H), lambda i: (0, i, 0))
_spec_cnt = pl.BlockSpec((NC, RB, 1), lambda i: (0, i, 0))
_spec_h = pl.BlockSpec((RB, H), lambda i: (i, 0))
_spec_gh = pl.BlockSpec((RB, 3 * H), lambda i: (i, 0))
_spec_wih = pl.BlockSpec((H, 3 * H), lambda i: (0, 0))
_spec_bih = pl.BlockSpec((8, 3 * H), lambda i: (0, 0))
_spec_wcat = pl.BlockSpec((H, 4 * H), lambda i: (0, 0))
_spec_bcat = pl.BlockSpec((8, 4 * H), lambda i: (0, 0))

_mm = pl.pallas_call(
    _mm_body,
    grid=(GRID,),
    in_specs=[_spec_h, _spec_wcat, _spec_bcat],
    out_specs=[_spec_h, _spec_gh],
    out_shape=[
        jax.ShapeDtypeStruct((N, H), jnp.float32),
        jax.ShapeDtypeStruct((N, 3 * H), jnp.float32),
    ],
)

_gru = pl.pallas_call(
    _gru_body,
    grid=(GRID,),
    in_specs=[_spec_part, _spec_cnt, _spec_gh, _spec_h, _spec_wih, _spec_bih],
    out_specs=_spec_h,
    out_shape=jax.ShapeDtypeStruct((N, H), jnp.float32),
)

_gmm = pl.pallas_call(
    _gmm_body,
    grid=(GRID,),
    in_specs=[_spec_part, _spec_cnt, _spec_gh, _spec_h, _spec_wih, _spec_bih,
              _spec_wcat, _spec_bcat],
    out_specs=[_spec_h, _spec_h, _spec_gh],
    out_shape=[
        jax.ShapeDtypeStruct((N, H), jnp.float32),
        jax.ShapeDtypeStruct((N, H), jnp.float32),
        jax.ShapeDtypeStruct((N, 3 * H), jnp.float32),
    ],
)


def kernel(node_states, edge_lists, W_msg, b_msg, w_ih, w_hh, b_ih, b_hh):
    f32 = jnp.float32
    el = edge_lists[0]
    src = el[:, 0].astype(jnp.int32)
    tgt = el[:, 1].astype(jnp.int32)
    pad = E_PAD - E
    pad_src = jnp.arange(pad, dtype=jnp.int32) % N
    pad_tgt = N + (jnp.arange(pad, dtype=jnp.int32) % (N_PAD - N))
    srcp = jnp.concatenate([src, pad_src]).reshape(NW, CH_PER_TILE, CHUNK)
    tgtp = jnp.concatenate([tgt, pad_tgt]).reshape(NW, CH_PER_TILE, CHUNK)

    zrows = jnp.zeros((ROWS_PER_TILE, H), f32)
    z1 = jnp.zeros((ROWS_PER_TILE,), f32)
    ones1 = jnp.ones((CHUNK,), f32)

    edge_k, cnt_k = _sc_kernels()
    cnt_part = cnt_k(tgtp, ones1, z1).reshape(NC, N_PAD, 1)

    wT_cat = jnp.concatenate([W_msg, w_hh], axis=0).T
    bc = jnp.broadcast_to(
        jnp.concatenate([b_msg, b_hh])[None, :], (8, 4 * H))
    wT_ih = w_ih.T
    bih = jnp.broadcast_to(b_ih[None, :], (8, 3 * H))

    h = node_states
    prop, gh = _mm(h, wT_cat, bc)
    for t in range(TIMESTEPS):
        part = edge_k(prop, srcp, tgtp, zrows)
        if t + 1 < TIMESTEPS:
            h, prop, gh = _gmm(part, cnt_part, gh, h, wT_ih, bih, wT_cat, bc)
        else:
            h = _gru(part, cnt_part, gh, h, wT_ih, bih)
    return h, node_states


# speedup vs baseline: 7.7623x; 1.0329x over previous
import functools

import jax
import jax.numpy as jnp
from jax import lax
from jax.experimental import pallas as pl
from jax.experimental.pallas import tpu as pltpu
from jax.experimental.pallas import tpu_sc as plsc

N = 10000
H = 128
E = 320000
TIMESTEPS = 4

NC = 2
NS = 16
NW = NC * NS
CHUNK = 128
CH_PER_TILE = 80
BLK = 40
NBLK = CH_PER_TILE // BLK
E_PAD = NW * CH_PER_TILE * CHUNK
N_PAD = 10240
ROWS_PER_TILE = N_PAD // NS

RB = 200
GRID = N // RB


def _edge_body(prop_hbm, src_hbm, tgt_hbm, zr_hbm, out_hbm, srca, tgta,
               rows0, rows1, acc, g0, g1, s0, s1):
    c = lax.axis_index("c")
    s = lax.axis_index("s")
    wid = s * NC + c
    row0 = s * ROWS_PER_TILE
    pltpu.sync_copy(zr_hbm, acc.at[pl.ds(row0, ROWS_PER_TILE)])
    plsc.subcore_barrier()

    bufs = [(rows0, g0, s0), (rows1, g1, s1)]

    @pl.loop(0, NBLK)
    def _blk(blk):
        pltpu.sync_copy(src_hbm.at[wid, pl.ds(blk * BLK, BLK)], srca)
        pltpu.sync_copy(tgt_hbm.at[wid, pl.ds(blk * BLK, BLK)], tgta)
        pltpu.async_copy(prop_hbm.at[srca.at[0]], rows0, g0)
        for k in range(BLK):
            rb, gs, ss = bufs[k % 2]
            pltpu.make_async_copy(prop_hbm.at[srca.at[k]], rb, gs).wait()
            if k + 1 < BLK:
                ob, og, osem = bufs[(k + 1) % 2]
                if k >= 1:
                    pltpu.make_async_copy(ob, acc.at[tgta.at[k - 1]],
                                          osem).wait()
                pltpu.async_copy(prop_hbm.at[srca.at[k + 1]], ob, og)
            pltpu.async_copy(rb, acc.at[tgta.at[k]], ss, add=True)
        rb, _, ss = bufs[(BLK - 2) % 2]
        pltpu.make_async_copy(rb, acc.at[tgta.at[BLK - 2]], ss).wait()
        rb, _, ss = bufs[(BLK - 1) % 2]
        pltpu.make_async_copy(rb, acc.at[tgta.at[BLK - 1]], ss).wait()

    plsc.subcore_barrier()
    pltpu.sync_copy(acc.at[pl.ds(row0, ROWS_PER_TILE)],
                    out_hbm.at[c, pl.ds(row0, ROWS_PER_TILE)])


def _cnt_body(tgt_hbm, ones_hbm, z1_hbm, out_hbm, tgta, ones, acc1, csem):
    c = lax.axis_index("c")
    s = lax.axis_index("s")
    wid = s * NC + c
    row0 = s * ROWS_PER_TILE
    pltpu.sync_copy(z1_hbm, acc1.at[pl.ds(row0, ROWS_PER_TILE)])
    pltpu.sync_copy(tgt_hbm.at[wid], tgta)
    pltpu.sync_copy(ones_hbm, ones)
    plsc.subcore_barrier()

    for i in range(CH_PER_TILE):
        pltpu.async_copy(ones, acc1.at[tgta.at[i]], csem, add=True)
    for i in range(CH_PER_TILE):
        pltpu.make_async_copy(ones, acc1.at[tgta.at[i]], csem).wait()
    plsc.subcore_barrier()
    pltpu.sync_copy(acc1.at[pl.ds(row0, ROWS_PER_TILE)],
                    out_hbm.at[c, pl.ds(row0, ROWS_PER_TILE)])


@functools.lru_cache(maxsize=None)
def _sc_kernels():
    mesh = plsc.VectorSubcoreMesh(core_axis_name="c", subcore_axis_name="s",
                                  num_cores=NC, num_subcores=NS)
    edge = pl.kernel(
        _edge_body,
        out_type=jax.ShapeDtypeStruct((NC, N_PAD, H), jnp.float32),
        mesh=mesh,
        scratch_types=[
            pltpu.VMEM((BLK, CHUNK), jnp.int32),
            pltpu.VMEM((BLK, CHUNK), jnp.int32),
            pltpu.VMEM((CHUNK, H), jnp.float32),
            pltpu.VMEM((CHUNK, H), jnp.float32),
            pltpu.VMEM_SHARED((N_PAD, H), jnp.float32),
            pltpu.SemaphoreType.DMA,
            pltpu.SemaphoreType.DMA,
            pltpu.SemaphoreType.DMA,
            pltpu.SemaphoreType.DMA,
        ],
    )
    cnt = pl.kernel(
        _cnt_body,
        out_type=jax.ShapeDtypeStruct((NC, N_PAD), jnp.float32),
        mesh=mesh,
        scratch_types=[
            pltpu.VMEM((CH_PER_TILE, CHUNK), jnp.int32),
            pltpu.VMEM((CHUNK,), jnp.float32),
            pltpu.VMEM_SHARED((N_PAD,), jnp.float32),
            pltpu.SemaphoreType.DMA,
        ],
    )
    return edge, cnt


def _gru_math(p_ref, cnt_ref, h, wT_ih_ref, bih_ref, wT_hh_ref, bhh_ref):
    p = p_ref[0] + p_ref[1]
    cnt = cnt_ref[0] + cnt_ref[1]
    x = p / jnp.maximum(cnt, 1.0)
    gi = jnp.dot(x, wT_ih_ref[...], preferred_element_type=jnp.float32)
    gi = gi + bih_ref[0:1, :]
    gh = jnp.dot(h, wT_hh_ref[...], preferred_element_type=jnp.float32)
    gh = gh + bhh_ref[0:1, :]
    r = jax.nn.sigmoid(gi[:, :H] + gh[:, :H])
    z = jax.nn.sigmoid(gi[:, H:2 * H] + gh[:, H:2 * H])
    n = jnp.tanh(gi[:, 2 * H:] + r * gh[:, 2 * H:])
    return (1.0 - z) * n + z * h


def _mm0_body(h_ref, wT_msg_ref, bmsg_ref, prop_ref):
    y = jnp.dot(h_ref[...], wT_msg_ref[...],
                preferred_element_type=jnp.float32)
    prop_ref[...] = y + bmsg_ref[0:1, :]


def _step_body(p_ref, cnt_ref, h_ref, wT_ih_ref, bih_ref, wT_hh_ref,
               bhh_ref, wT_msg_ref, bmsg_ref, hn_ref, prop_ref):
    hn = _gru_math(p_ref, cnt_ref, h_ref[...], wT_ih_ref, bih_ref,
                   wT_hh_ref, bhh_ref)
    hn_ref[...] = hn
    y = jnp.dot(hn, wT_msg_ref[...], preferred_element_type=jnp.float32)
    prop_ref[...] = y + bmsg_ref[0:1, :]


def _last_body(p_ref, cnt_ref, h_ref, wT_ih_ref, bih_ref, wT_hh_ref,
               bhh_ref, hn_ref):
    hn_ref[...] = _gru_math(p_ref, cnt_ref, h_ref[...], wT_ih_ref, bih_ref,
                            wT_hh_ref, bhh_ref)


_spec_part = pl.BlockSpec((NC, RB, H), lambda i: (0, i, 0))
_spec_cnt = pl.BlockSpec((NC, RB, 1), lambda i: (0, i, 0))
_spec_h = pl.BlockSpec((RB, H), lambda i: (i, 0))
_spec_w3 = pl.BlockSpec((H, 3 * H), lambda i: (0, 0))
_spec_b3 = pl.BlockSpec((8, 3 * H), lambda i: (0, 0))
_spec_w1 = pl.BlockSpec((H, H), lambda i: (0, 0))
_spec_b1 = pl.BlockSpec((8, H), lambda i: (0, 0))

_sds_h = jax.ShapeDtypeStruct((N, H), jnp.float32)

_mm0 = pl.pallas_call(
    _mm0_body,
    grid=(GRID,),
    in_specs=[_spec_h, _spec_w1, _spec_b1],
    out_specs=_spec_h,
    out_shape=_sds_h,
)

_step = pl.pallas_call(
    _step_body,
    grid=(GRID,),
    in_specs=[_spec_part, _spec_cnt, _spec_h, _spec_w3, _spec_b3, _spec_w3,
              _spec_b3, _spec_w1, _spec_b1],
    out_specs=[_spec_h, _spec_h],
    out_shape=[_sds_h, _sds_h],
)

_last = pl.pallas_call(
    _last_body,
    grid=(GRID,),
    in_specs=[_spec_part, _spec_cnt, _spec_h, _spec_w3, _spec_b3, _spec_w3,
              _spec_b3],
    out_specs=_spec_h,
    out_shape=_sds_h,
)


def kernel(node_states, edge_lists, W_msg, b_msg, w_ih, w_hh, b_ih, b_hh):
    f32 = jnp.float32
    el = edge_lists[0]
    src = el[:, 0].astype(jnp.int32)
    tgt = el[:, 1].astype(jnp.int32)
    pad = E_PAD - E
    pad_src = jnp.arange(pad, dtype=jnp.int32) % N
    pad_tgt = N + (jnp.arange(pad, dtype=jnp.int32) % (N_PAD - N))
    srcp = jnp.concatenate([src, pad_src]).reshape(NW, CH_PER_TILE, CHUNK)
    tgtp = jnp.concatenate([tgt, pad_tgt]).reshape(NW, CH_PER_TILE, CHUNK)

    zrows = jnp.zeros((ROWS_PER_TILE, H), f32)
    z1 = jnp.zeros((ROWS_PER_TILE,), f32)
    ones1 = jnp.ones((CHUNK,), f32)

    edge_k, cnt_k = _sc_kernels()
    cnt_part = cnt_k(tgtp, ones1, z1).reshape(NC, N_PAD, 1)

    wT_msg = W_msg.T
    bmsg = jnp.broadcast_to(b_msg[None, :], (8, H))
    wT_ih = w_ih.T
    bih = jnp.broadcast_to(b_ih[None, :], (8, 3 * H))
    wT_hh = w_hh.T
    bhh = jnp.broadcast_to(b_hh[None, :], (8, 3 * H))

    h = node_states
    prop = _mm0(h, wT_msg, bmsg)
    for t in range(TIMESTEPS):
        part = edge_k(prop, srcp, tgtp, zrows)
        if t + 1 < TIMESTEPS:
            h, prop = _step(part, cnt_part, h, wT_ih, bih, wT_hh, bhh,
                            wT_msg, bmsg)
        else:
            h = _last(part, cnt_part, h, wT_ih, bih, wT_hh, bhh)
    return h, node_states
